# baseline (speedup 1.0000x reference)
import jax
import jax.numpy as jnp
from jax import lax
from jax.experimental import pallas as pl
from jax.experimental.pallas import tpu as pltpu

VOCAB = 100000
EMBED_DIM = 16
BATCH = 1024

_VBLK = 1024
_LANES = 128


def _proj_body(idx_ref, tt_ref, wt_ref, b_ref, out_ref, lat_ref):
    @pl.when(pl.program_id(0) == 0)
    def _gather():
        lane_ids = lax.broadcasted_iota(jnp.int32, (EMBED_DIM, _LANES), 1)

        def tile_body(t, _):
            def lane_body(i, acc):
                c = idx_ref[t * _LANES + i]
                cb = pl.multiple_of((c // _LANES) * _LANES, _LANES)
                tile = tt_ref[:, pl.ds(cb, _LANES)]
                rolled = pltpu.roll(tile, i - (c - cb), axis=1)
                return jnp.where(lane_ids == i, rolled, acc)

            acc = lax.fori_loop(
                0, _LANES, lane_body,
                jnp.zeros((EMBED_DIM, _LANES), jnp.float32), unroll=8)
            lat_ref[:, pl.ds(pl.multiple_of(t * _LANES, _LANES), _LANES)] = acc
            return 0

        lax.fori_loop(0, BATCH // _LANES, tile_body, 0)

    bcol = lax.dot_general(
        b_ref[...], jnp.ones((1, 1), jnp.float32),
        dimension_numbers=(((0,), (0,)), ((), ())),
        preferred_element_type=jnp.float32,
    )
    out_ref[...] = lax.dot_general(
        wt_ref[...], lat_ref[...],
        dimension_numbers=(((0,), (0,)), ((), ())),
        preferred_element_type=jnp.float32,
    ) + bcol


def _project(idx, tableT, wt, brow):
    grid = (pl.cdiv(VOCAB, _VBLK),)
    return pl.pallas_call(
        _proj_body,
        grid=grid,
        in_specs=[
            pl.BlockSpec(memory_space=pltpu.SMEM),
            pl.BlockSpec((EMBED_DIM, VOCAB), lambda j: (0, 0)),
            pl.BlockSpec((EMBED_DIM, _VBLK), lambda j: (0, j)),
            pl.BlockSpec((1, _VBLK), lambda j: (0, j)),
        ],
        out_specs=pl.BlockSpec((_VBLK, BATCH), lambda j: (j, 0)),
        out_shape=jax.ShapeDtypeStruct((VOCAB, BATCH), jnp.float32),
        scratch_shapes=[pltpu.VMEM((EMBED_DIM, BATCH), jnp.float32)],
        compiler_params=pltpu.CompilerParams(
            dimension_semantics=("arbitrary",),
            vmem_limit_bytes=100 * 1024 * 1024,
        ),
    )(idx, tableT, wt, brow)


def kernel(inputs, emb_table, W, b):
    idx = inputs.astype(jnp.int32)
    return _project(idx, emb_table.T, W.T, b[None, :]).T


# speedup vs baseline: 1.1883x; 1.1883x over previous
import jax
import jax.numpy as jnp
from jax import lax
from jax.experimental import pallas as pl
from jax.experimental.pallas import tpu as pltpu

VOCAB = 100000
EMBED_DIM = 16
BATCH = 1024

_VBLK = 2048
_LANES = 128


def _proj_body(idx_ref, tt_ref, wt_ref, b_ref, out_ref, lat_ref):
    @pl.when(pl.program_id(0) == 0)
    def _gather():
        lane_ids = lax.broadcasted_iota(jnp.int32, (EMBED_DIM, _LANES), 1)

        def tile_body(t, _):
            def lane_body(i, acc):
                c = idx_ref[t * _LANES + i]
                cb = pl.multiple_of((c // _LANES) * _LANES, _LANES)
                tile = tt_ref[:, pl.ds(cb, _LANES)]
                rolled = pltpu.roll(tile, i - (c - cb), axis=1)
                return jnp.where(lane_ids == i, rolled, acc)

            acc = lax.fori_loop(
                0, _LANES, lane_body,
                jnp.zeros((EMBED_DIM, _LANES), jnp.float32), unroll=16)
            lat_ref[:, pl.ds(pl.multiple_of(t * _LANES, _LANES), _LANES)] = acc
            return 0

        lax.fori_loop(0, BATCH // _LANES, tile_body, 0)

    bcol = lax.dot_general(
        b_ref[...], jnp.ones((1, 1), jnp.float32),
        dimension_numbers=(((0,), (0,)), ((), ())),
        preferred_element_type=jnp.float32,
    )
    out_ref[...] = lax.dot_general(
        wt_ref[...], lat_ref[...],
        dimension_numbers=(((0,), (0,)), ((), ())),
        preferred_element_type=jnp.float32,
    ) + bcol


def _project(idx, tableT, wt, brow):
    grid = (pl.cdiv(VOCAB, _VBLK),)
    return pl.pallas_call(
        _proj_body,
        grid=grid,
        in_specs=[
            pl.BlockSpec(memory_space=pltpu.SMEM),
            pl.BlockSpec((EMBED_DIM, VOCAB), lambda j: (0, 0)),
            pl.BlockSpec((EMBED_DIM, _VBLK), lambda j: (0, j)),
            pl.BlockSpec((1, _VBLK), lambda j: (0, j)),
        ],
        out_specs=pl.BlockSpec((_VBLK, BATCH), lambda j: (j, 0)),
        out_shape=jax.ShapeDtypeStruct((VOCAB, BATCH), jnp.float32),
        scratch_shapes=[pltpu.VMEM((EMBED_DIM, BATCH), jnp.float32)],
        compiler_params=pltpu.CompilerParams(
            dimension_semantics=("arbitrary",),
            vmem_limit_bytes=100 * 1024 * 1024,
        ),
    )(idx, tableT, wt, brow)


def kernel(inputs, emb_table, W, b):
    idx = inputs.astype(jnp.int32)
    return _project(idx, emb_table.T, W.T, b[None, :]).T


# speedup vs baseline: 1.2003x; 1.0102x over previous
import jax
import jax.numpy as jnp
from jax import lax
from jax.experimental import pallas as pl
from jax.experimental.pallas import tpu as pltpu

VOCAB = 100000
EMBED_DIM = 16
BATCH = 1024

_VBLK = 2048
_LANES = 128


def _proj_body(idx_ref, tt_ref, wt_ref, b_ref, out_ref, lat_ref):
    @pl.when(pl.program_id(0) == 0)
    def _gather():
        lane_ids = lax.broadcasted_iota(jnp.int32, (EMBED_DIM, _LANES), 1)

        def tile_body(t, _):
            def lane_body(i, acc):
                c = idx_ref[t * _LANES + i]
                cb = pl.multiple_of((c // _LANES) * _LANES, _LANES)
                tile = tt_ref[:, pl.ds(cb, _LANES)]
                rolled = pltpu.roll(tile, i - (c - cb), axis=1)
                return jnp.where(lane_ids == i, rolled, acc)

            acc = lax.fori_loop(
                0, _LANES, lane_body,
                jnp.zeros((EMBED_DIM, _LANES), jnp.float32), unroll=32)
            lat_ref[:, pl.ds(pl.multiple_of(t * _LANES, _LANES), _LANES)] = acc
            return 0

        lax.fori_loop(0, BATCH // _LANES, tile_body, 0)

    bcol = lax.dot_general(
        b_ref[...], jnp.ones((1, 1), jnp.float32),
        dimension_numbers=(((0,), (0,)), ((), ())),
        preferred_element_type=jnp.float32,
    )
    out_ref[...] = lax.dot_general(
        wt_ref[...], lat_ref[...],
        dimension_numbers=(((0,), (0,)), ((), ())),
        preferred_element_type=jnp.float32,
    ) + bcol


def _project(idx, tableT, wt, brow):
    grid = (pl.cdiv(VOCAB, _VBLK),)
    return pl.pallas_call(
        _proj_body,
        grid=grid,
        in_specs=[
            pl.BlockSpec(memory_space=pltpu.SMEM),
            pl.BlockSpec((EMBED_DIM, VOCAB), lambda j: (0, 0)),
            pl.BlockSpec((EMBED_DIM, _VBLK), lambda j: (0, j)),
            pl.BlockSpec((1, _VBLK), lambda j: (0, j)),
        ],
        out_specs=pl.BlockSpec((_VBLK, BATCH), lambda j: (j, 0)),
        out_shape=jax.ShapeDtypeStruct((VOCAB, BATCH), jnp.float32),
        scratch_shapes=[pltpu.VMEM((EMBED_DIM, BATCH), jnp.float32)],
        compiler_params=pltpu.CompilerParams(
            dimension_semantics=("arbitrary",),
            vmem_limit_bytes=100 * 1024 * 1024,
        ),
    )(idx, tableT, wt, brow)


def kernel(inputs, emb_table, W, b):
    idx = inputs.astype(jnp.int32)
    return _project(idx, emb_table.T, W.T, b[None, :]).T


# speedup vs baseline: 1.2078x; 1.0062x over previous
import jax
import jax.numpy as jnp
from jax import lax
from jax.experimental import pallas as pl
from jax.experimental.pallas import tpu as pltpu

VOCAB = 100000
EMBED_DIM = 16
BATCH = 1024

_VBLK = 2048
_LANES = 128


def _proj_body(idx_ref, tt_ref, wt_ref, b_ref, out_ref, lat_ref):
    @pl.when(pl.program_id(0) == 0)
    def _gather():
        lane_ids = lax.broadcasted_iota(jnp.int32, (EMBED_DIM, _LANES), 1)

        def tile_body(t, _):
            def lane_body(i, acc):
                c = idx_ref[t * _LANES + i]
                cb = pl.multiple_of((c // _LANES) * _LANES, _LANES)
                tile = tt_ref[:, pl.ds(cb, _LANES)]
                rolled = pltpu.roll(tile, i - (c - cb), axis=1)
                return jnp.where(lane_ids == i, rolled, acc)

            acc = lax.fori_loop(
                0, _LANES, lane_body,
                jnp.zeros((EMBED_DIM, _LANES), jnp.float32), unroll=64)
            lat_ref[:, pl.ds(pl.multiple_of(t * _LANES, _LANES), _LANES)] = acc
            return 0

        lax.fori_loop(0, BATCH // _LANES, tile_body, 0)

    bcol = lax.dot_general(
        b_ref[...], jnp.ones((1, 1), jnp.float32),
        dimension_numbers=(((0,), (0,)), ((), ())),
        preferred_element_type=jnp.float32,
    )
    out_ref[...] = lax.dot_general(
        wt_ref[...], lat_ref[...],
        dimension_numbers=(((0,), (0,)), ((), ())),
        preferred_element_type=jnp.float32,
    ) + bcol


def _project(idx, tableT, wt, brow):
    grid = (pl.cdiv(VOCAB, _VBLK),)
    return pl.pallas_call(
        _proj_body,
        grid=grid,
        in_specs=[
            pl.BlockSpec(memory_space=pltpu.SMEM),
            pl.BlockSpec((EMBED_DIM, VOCAB), lambda j: (0, 0)),
            pl.BlockSpec((EMBED_DIM, _VBLK), lambda j: (0, j)),
            pl.BlockSpec((1, _VBLK), lambda j: (0, j)),
        ],
        out_specs=pl.BlockSpec((_VBLK, BATCH), lambda j: (j, 0)),
        out_shape=jax.ShapeDtypeStruct((VOCAB, BATCH), jnp.float32),
        scratch_shapes=[pltpu.VMEM((EMBED_DIM, BATCH), jnp.float32)],
        compiler_params=pltpu.CompilerParams(
            dimension_semantics=("arbitrary",),
            vmem_limit_bytes=100 * 1024 * 1024,
        ),
    )(idx, tableT, wt, brow)


def kernel(inputs, emb_table, W, b):
    idx = inputs.astype(jnp.int32)
    return _project(idx, emb_table.T, W.T, b[None, :]).T


# speedup vs baseline: 1.2138x; 1.0050x over previous
import jax
import jax.numpy as jnp
from jax import lax
from jax.experimental import pallas as pl
from jax.experimental.pallas import tpu as pltpu

VOCAB = 100000
EMBED_DIM = 16
BATCH = 1024

_VBLK = 2048
_LANES = 128


def _proj_body(idx_ref, tt_ref, wt_ref, b_ref, out_ref, lat_ref):
    @pl.when(pl.program_id(0) == 0)
    def _gather():
        lane_ids = lax.broadcasted_iota(jnp.int32, (EMBED_DIM, _LANES), 1)

        def tile_body(t, _):
            def lane_body(i, acc):
                c = idx_ref[t * _LANES + i]
                cb = pl.multiple_of((c // _LANES) * _LANES, _LANES)
                tile = tt_ref[:, pl.ds(cb, _LANES)]
                rolled = pltpu.roll(tile, i - (c - cb), axis=1)
                return jnp.where(lane_ids == i, rolled, acc)

            acc = lax.fori_loop(
                0, _LANES, lane_body,
                jnp.zeros((EMBED_DIM, _LANES), jnp.float32), unroll=128)
            lat_ref[:, pl.ds(pl.multiple_of(t * _LANES, _LANES), _LANES)] = acc
            return 0

        lax.fori_loop(0, BATCH // _LANES, tile_body, 0)

    bcol = lax.dot_general(
        b_ref[...], jnp.ones((1, 1), jnp.float32),
        dimension_numbers=(((0,), (0,)), ((), ())),
        preferred_element_type=jnp.float32,
    )
    out_ref[...] = lax.dot_general(
        wt_ref[...], lat_ref[...],
        dimension_numbers=(((0,), (0,)), ((), ())),
        preferred_element_type=jnp.float32,
    ) + bcol


def _project(idx, tableT, wt, brow):
    grid = (pl.cdiv(VOCAB, _VBLK),)
    return pl.pallas_call(
        _proj_body,
        grid=grid,
        in_specs=[
            pl.BlockSpec(memory_space=pltpu.SMEM),
            pl.BlockSpec((EMBED_DIM, VOCAB), lambda j: (0, 0)),
            pl.BlockSpec((EMBED_DIM, _VBLK), lambda j: (0, j)),
            pl.BlockSpec((1, _VBLK), lambda j: (0, j)),
        ],
        out_specs=pl.BlockSpec((_VBLK, BATCH), lambda j: (j, 0)),
        out_shape=jax.ShapeDtypeStruct((VOCAB, BATCH), jnp.float32),
        scratch_shapes=[pltpu.VMEM((EMBED_DIM, BATCH), jnp.float32)],
        compiler_params=pltpu.CompilerParams(
            dimension_semantics=("arbitrary",),
            vmem_limit_bytes=100 * 1024 * 1024,
        ),
    )(idx, tableT, wt, brow)


def kernel(inputs, emb_table, W, b):
    idx = inputs.astype(jnp.int32)
    return _project(idx, emb_table.T, W.T, b[None, :]).T


# speedup vs baseline: 1.2168x; 1.0025x over previous
import jax
import jax.numpy as jnp
from jax import lax
from jax.experimental import pallas as pl
from jax.experimental.pallas import tpu as pltpu

VOCAB = 100000
EMBED_DIM = 16
BATCH = 1024

_VBLK = 2048
_LANES = 128


def _proj_body(idx_ref, tt_ref, wt_ref, b_ref, out_ref, lat_ref):
    @pl.when(pl.program_id(0) == 0)
    def _gather():
        lane_ids = lax.broadcasted_iota(jnp.int32, (EMBED_DIM, _LANES), 1)

        def tile_body(t, _):
            def lane_body(i, acc):
                c = idx_ref[t * _LANES + i]
                cb = pl.multiple_of((c // _LANES) * _LANES, _LANES)
                tile = tt_ref[:, pl.ds(cb, _LANES)]
                rolled = pltpu.roll(tile, i - (c - cb), axis=1)
                return jnp.where(lane_ids == i, rolled, acc)

            acc = lax.fori_loop(
                0, _LANES, lane_body,
                jnp.zeros((EMBED_DIM, _LANES), jnp.float32), unroll=128)
            lat_ref[:, pl.ds(pl.multiple_of(t * _LANES, _LANES), _LANES)] = acc
            return 0

        lax.fori_loop(0, BATCH // _LANES, tile_body, 0, unroll=2)

    bcol = lax.dot_general(
        b_ref[...], jnp.ones((1, 1), jnp.float32),
        dimension_numbers=(((0,), (0,)), ((), ())),
        preferred_element_type=jnp.float32,
    )
    out_ref[...] = lax.dot_general(
        wt_ref[...], lat_ref[...],
        dimension_numbers=(((0,), (0,)), ((), ())),
        preferred_element_type=jnp.float32,
    ) + bcol


def _project(idx, tableT, wt, brow):
    grid = (pl.cdiv(VOCAB, _VBLK),)
    return pl.pallas_call(
        _proj_body,
        grid=grid,
        in_specs=[
            pl.BlockSpec(memory_space=pltpu.SMEM),
            pl.BlockSpec((EMBED_DIM, VOCAB), lambda j: (0, 0)),
            pl.BlockSpec((EMBED_DIM, _VBLK), lambda j: (0, j)),
            pl.BlockSpec((1, _VBLK), lambda j: (0, j)),
        ],
        out_specs=pl.BlockSpec((_VBLK, BATCH), lambda j: (j, 0)),
        out_shape=jax.ShapeDtypeStruct((VOCAB, BATCH), jnp.float32),
        scratch_shapes=[pltpu.VMEM((EMBED_DIM, BATCH), jnp.float32)],
        compiler_params=pltpu.CompilerParams(
            dimension_semantics=("arbitrary",),
            vmem_limit_bytes=100 * 1024 * 1024,
        ),
    )(idx, tableT, wt, brow)


def kernel(inputs, emb_table, W, b):
    idx = inputs.astype(jnp.int32)
    return _project(idx, emb_table.T, W.T, b[None, :]).T
